# baseline (speedup 1.0000x reference)
import functools

import jax
import jax.numpy as jnp
from jax import lax
from jax.experimental import pallas as pl
from jax.experimental.pallas import tpu as pltpu
from jax.experimental.pallas import tpu_sc as plsc

N_EMBD = 256
N_EXPERTS = 8
HIDDEN = 682
HID_PAD = 768
N_TOK = 8192
N_ENT = 2 * N_TOK
TILE = 128
CAP = N_ENT + N_EXPERTS * TILE
N_TILES = CAP // TILE
NW = 32


def _shift_down(a, k):
    return jnp.pad(a, ((k, 0), (0, 0)))[: a.shape[0], :]


def _plan_kernel(x_ref, wr_ref, pos_ref, wn_ref, seg_ref):
    x = x_ref[...]
    logits = lax.dot_general(
        x, wr_ref[...], (((1,), (1,)), ((), ())),
        preferred_element_type=jnp.float32)
    m = jnp.max(logits, axis=-1, keepdims=True)
    unnorm = jnp.exp(logits - m)
    probs = unnorm / jnp.sum(unnorm, axis=-1, keepdims=True)
    eidx = lax.broadcasted_iota(jnp.int32, probs.shape, 1)
    p1 = jnp.max(probs, axis=-1, keepdims=True)
    i1 = jnp.min(jnp.where(probs == p1, eidx, N_EXPERTS), axis=-1,
                 keepdims=True)
    probs_m = jnp.where(eidx == i1, -1.0, probs)
    p2 = jnp.max(probs_m, axis=-1, keepdims=True)
    i2 = jnp.min(jnp.where(probs_m == p2, eidx, N_EXPERTS), axis=-1,
                 keepdims=True)
    denom = p1 + p2 + 1e-9
    w1n = p1 / denom
    w2n = p2 / denom

    oh1 = (eidx == i1).astype(jnp.float32)
    oh2 = (eidx == i2).astype(jnp.float32)
    onehot = jnp.concatenate([oh1, oh2], axis=0)

    inc = onehot
    k = 1
    while k < N_ENT:
        inc = inc + _shift_down(inc, k)
        k *= 2
    excl = inc - onehot
    counts = jnp.sum(onehot, axis=0, keepdims=True)
    padded = jnp.floor((counts + (TILE - 1)) / TILE) * TILE
    t = padded
    for k in (1, 2, 4):
        t = t + jnp.pad(t, ((0, 0), (k, 0)))[:, :N_EXPERTS]
    seg_start = t - padded
    seg_end = t

    rank = jnp.sum(excl * onehot, axis=-1)
    base = jnp.sum(onehot * seg_start, axis=-1)
    pos = (base + rank).astype(jnp.int32)
    pos_ref[...] = pos.reshape(2, N_TOK)
    wn_ref[...] = jnp.concatenate([w1n, w2n], axis=0).reshape(2, N_TOK)
    seg_ref[...] = jnp.concatenate(
        [seg_start, seg_end], axis=0).astype(jnp.int32)


def _dispatch_kernel(pos_hbm, wn_hbm, flat_hbm, xs_hbm, wrow_hbm,
                     pos_v, wn_v, perm_v, wrow_v, rows_v, sem):
    wid = lax.axis_index("s") * 2 + lax.axis_index("c")
    pltpu.sync_copy(pos_hbm, pos_v)
    pltpu.sync_copy(wn_hbm, wn_v)

    zero_i = jnp.zeros((16,), jnp.int32)
    zero_f = jnp.zeros((16,), jnp.float32)

    def init_body(i, _):
        perm_v[pl.ds(i * 16, 16)] = zero_i
        wrow_v[pl.ds(i * 16, 16)] = zero_f
        return ()

    lax.fori_loop(0, CAP // 16, init_body, (), unroll=8)

    lane = lax.iota(jnp.int32, 16)

    def scat_body(i, _):
        s = i // (N_TOK // 16)
        j = i - s * (N_TOK // 16)
        idx = pos_v[s, pl.ds(j * 16, 16)]
        tok = j * 16 + lane
        plsc.store_scatter(perm_v, [idx], tok)
        plsc.store_scatter(wrow_v, [idx], wn_v[s, pl.ds(j * 16, 16)])
        return ()

    lax.fori_loop(0, N_ENT // 16, scat_body, (), unroll=8)

    def do_tile(tile_id):
        base = tile_id * TILE
        cp = pltpu.async_copy(
            flat_hbm.at[perm_v.at[pl.ds(base, TILE)]], rows_v, sem)
        cp.wait()
        pltpu.sync_copy(rows_v, xs_hbm.at[pl.ds(base, TILE)])
        pltpu.sync_copy(wrow_v.at[pl.ds(base, TILE)], wrow_hbm.at[tile_id])

    for j in range(4):
        do_tile(wid + NW * j)

    @pl.when(wid + NW * 4 < N_TILES)
    def _():
        do_tile(wid + NW * 4)


def _expert_kernel(te_ref, xs_ref, wrow_ref, w1_ref, w3_ref, w2_ref,
                   ex_ref):
    xb = xs_ref[...]
    g = lax.dot_general(
        xb, w1_ref[0], (((1,), (1,)), ((), ())),
        preferred_element_type=jnp.float32)
    u = lax.dot_general(
        xb, w3_ref[0], (((1,), (1,)), ((), ())),
        preferred_element_type=jnp.float32)
    h = (g * jax.nn.sigmoid(g)) * u
    ex = lax.dot_general(
        h, w2_ref[0], (((1,), (1,)), ((), ())),
        preferred_element_type=jnp.float32)
    ex_ref[...] = ex * wrow_ref[0, 0][:, None]


def _combine_kernel(pos_hbm, ex_hbm, out_hbm,
                    idx1_v, idx2_v, r1_v, r2_v, sem1, sem2):
    wid = lax.axis_index("s") * 2 + lax.axis_index("c")
    for chunk in range(2):
        tok0 = wid * (N_TOK // NW) + chunk * TILE
        pltpu.sync_copy(pos_hbm.at[0, pl.ds(tok0, TILE)], idx1_v)
        pltpu.sync_copy(pos_hbm.at[1, pl.ds(tok0, TILE)], idx2_v)
        c1 = pltpu.async_copy(ex_hbm.at[idx1_v], r1_v, sem1)
        c2 = pltpu.async_copy(ex_hbm.at[idx2_v], r2_v, sem2)
        c1.wait()
        c2.wait()

        def add_body(t, _):
            for c in range(N_EMBD // 16):
                sl = pl.ds(c * 16, 16)
                r1_v[t, sl] = r1_v[t, sl] + r2_v[t, sl]
            return ()

        lax.fori_loop(0, TILE, add_body, ())
        pltpu.sync_copy(r1_v, out_hbm.at[pl.ds(tok0, TILE)])


def kernel(x, W1, W2, W3, Wr):
    B, T, C = x.shape
    flat = x.reshape(-1, C)
    pad = HID_PAD - HIDDEN
    W1p = jnp.pad(W1, ((0, 0), (0, pad), (0, 0)))
    W3p = jnp.pad(W3, ((0, 0), (0, pad), (0, 0)))
    W2p = jnp.pad(W2, ((0, 0), (0, 0), (0, pad)))

    pos, wn, seg = pl.pallas_call(
        _plan_kernel,
        in_specs=[
            pl.BlockSpec((N_TOK, C), lambda: (0, 0)),
            pl.BlockSpec((N_EXPERTS, C), lambda: (0, 0)),
        ],
        out_specs=[
            pl.BlockSpec((2, N_TOK), lambda: (0, 0)),
            pl.BlockSpec((2, N_TOK), lambda: (0, 0)),
            pl.BlockSpec((2, N_EXPERTS), lambda: (0, 0)),
        ],
        out_shape=[
            jax.ShapeDtypeStruct((2, N_TOK), jnp.int32),
            jax.ShapeDtypeStruct((2, N_TOK), jnp.float32),
            jax.ShapeDtypeStruct((2, N_EXPERTS), jnp.int32),
        ],
    )(flat, Wr)

    seg_end = seg[1]
    tile_id = jnp.arange(N_TILES, dtype=jnp.int32) * TILE
    te = jnp.sum((tile_id[:, None] >= seg_end[None, :]).astype(jnp.int32),
                 axis=1)
    te = jnp.clip(te, 0, N_EXPERTS - 1)

    mesh = plsc.VectorSubcoreMesh(core_axis_name="c", subcore_axis_name="s",
                                  num_cores=2, num_subcores=16)
    xs, wrow = pl.kernel(
        _dispatch_kernel,
        out_type=[
            jax.ShapeDtypeStruct((CAP, C), jnp.float32),
            jax.ShapeDtypeStruct((N_TILES, TILE), jnp.float32),
        ],
        mesh=mesh,
        scratch_types=[
            pltpu.VMEM((2, N_TOK), jnp.int32),
            pltpu.VMEM((2, N_TOK), jnp.float32),
            pltpu.VMEM((CAP,), jnp.int32),
            pltpu.VMEM((CAP,), jnp.float32),
            pltpu.VMEM((TILE, C), jnp.float32),
            pltpu.SemaphoreType.DMA,
        ],
        compiler_params=pltpu.CompilerParams(needs_layout_passes=False),
    )(pos, wn, flat)

    grid_spec = pltpu.PrefetchScalarGridSpec(
        num_scalar_prefetch=1,
        grid=(N_TILES,),
        in_specs=[
            pl.BlockSpec((TILE, C), lambda i, te_ref: (i, 0)),
            pl.BlockSpec((1, 1, TILE), lambda i, te_ref: (i, 0, 0)),
            pl.BlockSpec((1, HID_PAD, C),
                         lambda i, te_ref: (te_ref[i], 0, 0)),
            pl.BlockSpec((1, HID_PAD, C),
                         lambda i, te_ref: (te_ref[i], 0, 0)),
            pl.BlockSpec((1, C, HID_PAD),
                         lambda i, te_ref: (te_ref[i], 0, 0)),
        ],
        out_specs=pl.BlockSpec((TILE, C), lambda i, te_ref: (i, 0)),
    )
    ex = pl.pallas_call(
        _expert_kernel,
        grid_spec=grid_spec,
        out_shape=jax.ShapeDtypeStruct((CAP, C), jnp.float32),
        compiler_params=pltpu.CompilerParams(
            dimension_semantics=("arbitrary",),
        ),
    )(te, xs, wrow.reshape(N_TILES, 1, TILE), W1p, W3p, W2p)

    out = pl.kernel(
        _combine_kernel,
        out_type=jax.ShapeDtypeStruct((N_TOK, C), jnp.float32),
        mesh=mesh,
        scratch_types=[
            pltpu.VMEM((TILE,), jnp.int32),
            pltpu.VMEM((TILE,), jnp.int32),
            pltpu.VMEM((TILE, C), jnp.float32),
            pltpu.VMEM((TILE, C), jnp.float32),
            pltpu.SemaphoreType.DMA,
            pltpu.SemaphoreType.DMA,
        ],
        compiler_params=pltpu.CompilerParams(needs_layout_passes=False),
    )(pos, ex)

    return out.reshape(B, T, C)


# speedup vs baseline: 1.1407x; 1.1407x over previous
import functools

import jax
import jax.numpy as jnp
from jax import lax
from jax.experimental import pallas as pl
from jax.experimental.pallas import tpu as pltpu
from jax.experimental.pallas import tpu_sc as plsc

N_EMBD = 256
N_EXPERTS = 8
HIDDEN = 682
HID_PAD = 768
N_TOK = 8192
N_ENT = 2 * N_TOK
TILE = 256
CAP = N_ENT + N_EXPERTS * TILE
N_TILES = CAP // TILE
WIN = 128
N_WIN = CAP // WIN
NW = 32
NSUB = 16
EPS = 1024


def _shift_rows(a, k):
    return jnp.pad(a, ((k, 0), (0, 0)))[: a.shape[0], :]


def _shift_lanes(a, k):
    return jnp.pad(a, ((0, 0), (k, 0)))[:, : a.shape[1]]


def _plan_kernel(x_ref, wr_ref, pos_ref, wn_ref, seg_ref):
    x = x_ref[...]
    logits_t = lax.dot_general(
        wr_ref[...], x, (((1,), (1,)), ((), ())),
        preferred_element_type=jnp.float32)
    m = jnp.max(logits_t, axis=0, keepdims=True)
    unnorm = jnp.exp(logits_t - m)
    probs = unnorm / jnp.sum(unnorm, axis=0, keepdims=True)
    eidx = lax.broadcasted_iota(jnp.int32, probs.shape, 0)
    p1 = jnp.max(probs, axis=0, keepdims=True)
    i1 = jnp.min(jnp.where(probs == p1, eidx, N_EXPERTS), axis=0,
                 keepdims=True)
    probs_m = jnp.where(eidx == i1, -1.0, probs)
    p2 = jnp.max(probs_m, axis=0, keepdims=True)
    i2 = jnp.min(jnp.where(probs_m == p2, eidx, N_EXPERTS), axis=0,
                 keepdims=True)
    denom = p1 + p2 + 1e-9
    w1n = p1 / denom
    w2n = p2 / denom

    oh1 = (eidx == i1).astype(jnp.float32)
    oh2 = (eidx == i2).astype(jnp.float32)
    oht = jnp.concatenate([oh1, oh2], axis=1)

    inc = oht
    k = 1
    while k < N_ENT:
        inc = inc + _shift_lanes(inc, k)
        k *= 2
    excl = inc - oht

    counts_i = inc[:, N_ENT - 1:N_ENT].astype(jnp.int32)
    padded = ((counts_i + TILE - 1) // TILE) * TILE
    t = padded
    for k in (1, 2, 4):
        t = t + _shift_rows(t, k)
    seg_start = t - padded
    seg_end = t

    rank = jnp.sum(excl * oht, axis=0, keepdims=True)
    base = jnp.sum(oht * seg_start.astype(jnp.float32), axis=0,
                   keepdims=True)
    pos_ref[...] = (rank + base).astype(jnp.int32)
    wn_ref[...] = jnp.concatenate([w1n, w2n], axis=1)
    seg_ref[...] = jnp.concatenate([seg_start, seg_end], axis=1)


def _dispatch_kernel(pos_hbm, wn_hbm, flat_hbm, xs_hbm, wrow_hbm,
                     idx_v, wn_v, vals_v, zb_i, zb_f,
                     winp_v, winw_v, rows_v, perm_sh, wrow_sh, sem):
    cid = lax.axis_index("c")
    sid = lax.axis_index("s")
    wid = sid * 2 + cid

    pltpu.sync_copy(pos_hbm.at[sid], idx_v)
    pltpu.sync_copy(wn_hbm.at[sid], wn_v)

    zeros_i = jnp.zeros((16,), jnp.int32)
    zeros_f = jnp.zeros((16,), jnp.float32)
    for g in range(CAP // NSUB // 16):
        zb_i[pl.ds(g * 16, 16)] = zeros_i
        zb_f[pl.ds(g * 16, 16)] = zeros_f
    pltpu.sync_copy(zb_i, perm_sh.at[pl.ds(sid * (CAP // NSUB), CAP // NSUB)])
    pltpu.sync_copy(zb_f, wrow_sh.at[pl.ds(sid * (CAP // NSUB), CAP // NSUB)])

    lane = lax.iota(jnp.int32, 16)
    for j in range(8):
        for g in range(8):
            ent = sid * EPS + j * 128 + g * 16 + lane
            tok = ent - jnp.where(ent >= N_TOK, N_TOK, 0)
            vals_v[j, pl.ds(g * 16, 16)] = tok

    plsc.subcore_barrier()

    for j in range(8):
        pltpu.sync_copy(vals_v.at[j], perm_sh.at[idx_v.at[j]], add=True)
        pltpu.sync_copy(wn_v.at[j], wrow_sh.at[idx_v.at[j]], add=True)

    plsc.subcore_barrier()

    def do_win(win_id):
        base = win_id * WIN
        pltpu.sync_copy(perm_sh.at[pl.ds(base, WIN)], winp_v)
        pltpu.sync_copy(wrow_sh.at[pl.ds(base, WIN)], winw_v)
        cp = pltpu.async_copy(flat_hbm.at[winp_v], rows_v, sem)
        cp.wait()
        pltpu.sync_copy(rows_v, xs_hbm.at[pl.ds(base, WIN)])
        pltpu.sync_copy(winw_v, wrow_hbm.at[win_id])

    for j in range(4):
        do_win(wid + NW * j)

    @pl.when(wid + NW * 4 < N_WIN)
    def _():
        do_win(wid + NW * 4)


def _expert_kernel(te_ref, xs_ref, wrow_ref, w1_ref, w3_ref, w2_ref,
                   ex_ref):
    xb = xs_ref[...].astype(jnp.bfloat16)
    g = lax.dot_general(
        xb, w1_ref[0], (((1,), (1,)), ((), ())),
        preferred_element_type=jnp.float32)
    u = lax.dot_general(
        xb, w3_ref[0], (((1,), (1,)), ((), ())),
        preferred_element_type=jnp.float32)
    h = (g * jax.nn.sigmoid(g)) * u
    ex = lax.dot_general(
        h.astype(jnp.bfloat16), w2_ref[0], (((1,), (1,)), ((), ())),
        preferred_element_type=jnp.float32)
    ex_ref[...] = ex * wrow_ref[0, 0][:, None]


def _combine_kernel(pos_hbm, ex_hbm, out_hbm,
                    idx1_v, idx2_v, r1_v, r2_v, sem1, sem2):
    cid = lax.axis_index("c")
    sid = lax.axis_index("s")
    wid = sid * 2 + cid
    for chunk in range(2):
        tok0 = wid * (N_TOK // NW) + chunk * 128
        pltpu.sync_copy(pos_hbm.at[0, pl.ds(tok0, 128)], idx1_v)
        pltpu.sync_copy(pos_hbm.at[1, pl.ds(tok0, 128)], idx2_v)
        c1 = pltpu.async_copy(ex_hbm.at[idx1_v], r1_v, sem1)
        c2 = pltpu.async_copy(ex_hbm.at[idx2_v], r2_v, sem2)
        c1.wait()
        c2.wait()

        def add_body(t, _):
            for c in range(N_EMBD // 16):
                sl = pl.ds(c * 16, 16)
                r1_v[t, sl] = r1_v[t, sl] + r2_v[t, sl]
            return ()

        lax.fori_loop(0, 128, add_body, ())
        pltpu.sync_copy(r1_v, out_hbm.at[pl.ds(tok0, 128)])


def kernel(x, W1, W2, W3, Wr):
    B, T, C = x.shape
    flat = x.reshape(-1, C)
    pad = HID_PAD - HIDDEN
    W1p = jnp.pad(W1, ((0, 0), (0, pad), (0, 0))).astype(jnp.bfloat16)
    W3p = jnp.pad(W3, ((0, 0), (0, pad), (0, 0))).astype(jnp.bfloat16)
    W2p = jnp.pad(W2, ((0, 0), (0, 0), (0, pad))).astype(jnp.bfloat16)

    pos_pk, wn_pk, seg = pl.pallas_call(
        _plan_kernel,
        in_specs=[
            pl.BlockSpec((N_TOK, C), lambda: (0, 0)),
            pl.BlockSpec((N_EXPERTS, C), lambda: (0, 0)),
        ],
        out_specs=[
            pl.BlockSpec((1, N_ENT), lambda: (0, 0)),
            pl.BlockSpec((1, N_ENT), lambda: (0, 0)),
            pl.BlockSpec((N_EXPERTS, 2), lambda: (0, 0)),
        ],
        out_shape=[
            jax.ShapeDtypeStruct((1, N_ENT), jnp.int32),
            jax.ShapeDtypeStruct((1, N_ENT), jnp.float32),
            jax.ShapeDtypeStruct((N_EXPERTS, 2), jnp.int32),
        ],
    )(flat, Wr)

    pos3d = pos_pk.reshape(NSUB, 8, 128)
    wn3d = wn_pk.reshape(NSUB, 8, 128)
    pos = pos_pk.reshape(2, N_TOK)

    seg_end = seg[:, 1]
    tile_base = jnp.arange(N_TILES, dtype=jnp.int32) * TILE
    te = jnp.sum((tile_base[:, None] >= seg_end[None, :]).astype(jnp.int32),
                 axis=1)
    te = jnp.clip(te, 0, N_EXPERTS - 1)

    mesh = plsc.VectorSubcoreMesh(core_axis_name="c", subcore_axis_name="s",
                                  num_cores=2, num_subcores=16)
    xs, wrow = pl.kernel(
        _dispatch_kernel,
        out_type=[
            jax.ShapeDtypeStruct((CAP, C), jnp.float32),
            jax.ShapeDtypeStruct((N_WIN, WIN), jnp.float32),
        ],
        mesh=mesh,
        scratch_types=[
            pltpu.VMEM((8, 128), jnp.int32),
            pltpu.VMEM((8, 128), jnp.float32),
            pltpu.VMEM((8, 128), jnp.int32),
            pltpu.VMEM((CAP // NSUB,), jnp.int32),
            pltpu.VMEM((CAP // NSUB,), jnp.float32),
            pltpu.VMEM((WIN,), jnp.int32),
            pltpu.VMEM((WIN,), jnp.float32),
            pltpu.VMEM((WIN, C), jnp.float32),
            pltpu.VMEM_SHARED((CAP,), jnp.int32),
            pltpu.VMEM_SHARED((CAP,), jnp.float32),
            pltpu.SemaphoreType.DMA,
        ],
        compiler_params=pltpu.CompilerParams(needs_layout_passes=False),
    )(pos3d, wn3d, flat)

    grid_spec = pltpu.PrefetchScalarGridSpec(
        num_scalar_prefetch=1,
        grid=(N_TILES,),
        in_specs=[
            pl.BlockSpec((TILE, C), lambda i, te_ref: (i, 0)),
            pl.BlockSpec((1, 1, TILE), lambda i, te_ref: (i, 0, 0)),
            pl.BlockSpec((1, HID_PAD, C),
                         lambda i, te_ref: (te_ref[i], 0, 0)),
            pl.BlockSpec((1, HID_PAD, C),
                         lambda i, te_ref: (te_ref[i], 0, 0)),
            pl.BlockSpec((1, C, HID_PAD),
                         lambda i, te_ref: (te_ref[i], 0, 0)),
        ],
        out_specs=pl.BlockSpec((TILE, C), lambda i, te_ref: (i, 0)),
    )
    ex = pl.pallas_call(
        _expert_kernel,
        grid_spec=grid_spec,
        out_shape=jax.ShapeDtypeStruct((CAP, C), jnp.float32),
        compiler_params=pltpu.CompilerParams(
            dimension_semantics=("arbitrary",),
        ),
    )(te, xs, wrow.reshape(N_TILES, 1, TILE), W1p, W3p, W2p)

    out = pl.kernel(
        _combine_kernel,
        out_type=jax.ShapeDtypeStruct((N_TOK, C), jnp.float32),
        mesh=mesh,
        scratch_types=[
            pltpu.VMEM((128,), jnp.int32),
            pltpu.VMEM((128,), jnp.int32),
            pltpu.VMEM((128, C), jnp.float32),
            pltpu.VMEM((128, C), jnp.float32),
            pltpu.SemaphoreType.DMA,
            pltpu.SemaphoreType.DMA,
        ],
        compiler_params=pltpu.CompilerParams(needs_layout_passes=False),
    )(pos, ex)

    return out.reshape(B, T, C)


# speedup vs baseline: 1.4807x; 1.2981x over previous
import functools

import jax
import jax.numpy as jnp
from jax import lax
from jax.experimental import pallas as pl
from jax.experimental.pallas import tpu as pltpu
from jax.experimental.pallas import tpu_sc as plsc

N_EMBD = 256
N_EXPERTS = 8
HIDDEN = 682
HID_PAD = 768
N_TOK = 8192
N_ENT = 2 * N_TOK
TILE = 256
CAP = N_ENT + N_EXPERTS * TILE
N_TILES = CAP // TILE
WIN = 128
N_WIN = CAP // WIN
NW = 32
NSUB = 16
EPS = 1024


def _shift_rows(a, k):
    return jnp.pad(a, ((k, 0), (0, 0)))[: a.shape[0], :]


def _shift_lanes(a, k):
    return jnp.pad(a, ((0, 0), (k, 0)))[:, : a.shape[1]]


def _plan_kernel(x_ref, wr_ref, pos_ref, wn_ref, seg_ref):
    x = x_ref[...]
    logits_t = lax.dot_general(
        wr_ref[...], x, (((1,), (1,)), ((), ())),
        preferred_element_type=jnp.float32)
    m = jnp.max(logits_t, axis=0, keepdims=True)
    unnorm = jnp.exp(logits_t - m)
    probs = unnorm / jnp.sum(unnorm, axis=0, keepdims=True)
    eidx = lax.broadcasted_iota(jnp.int32, probs.shape, 0)
    p1 = jnp.max(probs, axis=0, keepdims=True)
    i1 = jnp.min(jnp.where(probs == p1, eidx, N_EXPERTS), axis=0,
                 keepdims=True)
    probs_m = jnp.where(eidx == i1, -1.0, probs)
    p2 = jnp.max(probs_m, axis=0, keepdims=True)
    i2 = jnp.min(jnp.where(probs_m == p2, eidx, N_EXPERTS), axis=0,
                 keepdims=True)
    denom = p1 + p2 + 1e-9
    w1n = p1 / denom
    w2n = p2 / denom

    oh1 = (eidx == i1).astype(jnp.float32)
    oh2 = (eidx == i2).astype(jnp.float32)
    oht = jnp.concatenate([oh1, oh2], axis=1)

    inc = oht
    k = 1
    while k < N_ENT:
        inc = inc + _shift_lanes(inc, k)
        k *= 2
    excl = inc - oht

    counts_i = inc[:, N_ENT - 1:N_ENT].astype(jnp.int32)
    padded = ((counts_i + TILE - 1) // TILE) * TILE
    t = padded
    for k in (1, 2, 4):
        t = t + _shift_rows(t, k)
    seg_start = t - padded
    seg_end = t

    rank = jnp.sum(excl * oht, axis=0, keepdims=True)
    base = jnp.sum(oht * seg_start.astype(jnp.float32), axis=0,
                   keepdims=True)
    pos_ref[...] = (rank + base).astype(jnp.int32)
    wn_ref[...] = jnp.concatenate([w1n, w2n], axis=1)
    seg_ref[...] = jnp.concatenate([seg_start, seg_end], axis=1)


def _dispatch_kernel(pos_hbm, wn_hbm, flat_hbm, xs_hbm, wrow_hbm,
                     idx_v, wnv_v, rows_v, sem):
    cid = lax.axis_index("c")
    sid = lax.axis_index("s")
    wid = sid * 2 + cid
    tpw = N_TOK // NW

    pltpu.sync_copy(flat_hbm.at[pl.ds(wid * tpw, tpw)], rows_v)
    pltpu.sync_copy(pos_hbm.at[pl.ds(2 * wid, 2)], idx_v.at[pl.ds(0, 2)])
    pltpu.sync_copy(pos_hbm.at[pl.ds(N_TOK // 128 + 2 * wid, 2)],
                    idx_v.at[pl.ds(2, 2)])
    pltpu.sync_copy(wn_hbm.at[pl.ds(2 * wid, 2)], wnv_v.at[pl.ds(0, 2)])
    pltpu.sync_copy(wn_hbm.at[pl.ds(N_TOK // 128 + 2 * wid, 2)],
                    wnv_v.at[pl.ds(2, 2)])

    cps = []
    for r0, irow in ((0, 0), (128, 1), (0, 2), (128, 3)):
        cps.append(pltpu.async_copy(
            rows_v.at[pl.ds(r0, 128)], xs_hbm.at[idx_v.at[irow]], sem))
    for cp in cps:
        cp.wait()
    for irow in range(4):
        pltpu.sync_copy(wnv_v.at[irow], wrow_hbm.at[idx_v.at[irow]])


def _expert_kernel(te_ref, xs_ref, wrow_ref, w1_ref, w3_ref, w2_ref,
                   ex_ref):
    xb = xs_ref[...].astype(jnp.bfloat16)
    g = lax.dot_general(
        xb, w1_ref[0], (((1,), (1,)), ((), ())),
        preferred_element_type=jnp.float32)
    u = lax.dot_general(
        xb, w3_ref[0], (((1,), (1,)), ((), ())),
        preferred_element_type=jnp.float32)
    h = (g * jax.nn.sigmoid(g)) * u
    ex = lax.dot_general(
        h.astype(jnp.bfloat16), w2_ref[0], (((1,), (1,)), ((), ())),
        preferred_element_type=jnp.float32)
    ex_ref[...] = ex * wrow_ref[0, 0][:, None]


def _combine_kernel(pos_hbm, ex_hbm, out_hbm,
                    idx1_v, idx2_v, r1_v, r2_v, sem1, sem2):
    cid = lax.axis_index("c")
    sid = lax.axis_index("s")
    wid = sid * 2 + cid
    for chunk in range(2):
        tok0 = wid * (N_TOK // NW) + chunk * 128
        pltpu.sync_copy(pos_hbm.at[0, pl.ds(tok0, 128)], idx1_v)
        pltpu.sync_copy(pos_hbm.at[1, pl.ds(tok0, 128)], idx2_v)
        c1 = pltpu.async_copy(ex_hbm.at[idx1_v], r1_v, sem1)
        c2 = pltpu.async_copy(ex_hbm.at[idx2_v], r2_v, sem2)
        c1.wait()
        c2.wait()

        def add_body(t, _):
            for c in range(N_EMBD // 16):
                sl = pl.ds(c * 16, 16)
                r1_v[t, sl] = r1_v[t, sl] + r2_v[t, sl]
            return ()

        lax.fori_loop(0, 128, add_body, ())
        pltpu.sync_copy(r1_v, out_hbm.at[pl.ds(tok0, 128)])


def kernel(x, W1, W2, W3, Wr):
    B, T, C = x.shape
    flat = x.reshape(-1, C)
    pad = HID_PAD - HIDDEN
    W1p = jnp.pad(W1, ((0, 0), (0, pad), (0, 0))).astype(jnp.bfloat16)
    W3p = jnp.pad(W3, ((0, 0), (0, pad), (0, 0))).astype(jnp.bfloat16)
    W2p = jnp.pad(W2, ((0, 0), (0, 0), (0, pad))).astype(jnp.bfloat16)

    pos_pk, wn_pk, seg = pl.pallas_call(
        _plan_kernel,
        in_specs=[
            pl.BlockSpec((N_TOK, C), lambda: (0, 0)),
            pl.BlockSpec((N_EXPERTS, C), lambda: (0, 0)),
        ],
        out_specs=[
            pl.BlockSpec((1, N_ENT), lambda: (0, 0)),
            pl.BlockSpec((1, N_ENT), lambda: (0, 0)),
            pl.BlockSpec((N_EXPERTS, 2), lambda: (0, 0)),
        ],
        out_shape=[
            jax.ShapeDtypeStruct((1, N_ENT), jnp.int32),
            jax.ShapeDtypeStruct((1, N_ENT), jnp.float32),
            jax.ShapeDtypeStruct((N_EXPERTS, 2), jnp.int32),
        ],
    )(flat, Wr)

    pos2d = pos_pk.reshape(N_ENT // 128, 128)
    wn2d = wn_pk.reshape(N_ENT // 128, 128)
    pos = pos_pk.reshape(2, N_TOK)

    seg_end = seg[:, 1]
    tile_base = jnp.arange(N_TILES, dtype=jnp.int32) * TILE
    te = jnp.sum((tile_base[:, None] >= seg_end[None, :]).astype(jnp.int32),
                 axis=1)
    te = jnp.clip(te, 0, N_EXPERTS - 1)

    mesh = plsc.VectorSubcoreMesh(core_axis_name="c", subcore_axis_name="s",
                                  num_cores=2, num_subcores=16)
    xs, wrow = pl.kernel(
        _dispatch_kernel,
        out_type=[
            jax.ShapeDtypeStruct((CAP, C), jnp.float32),
            jax.ShapeDtypeStruct((CAP,), jnp.float32),
        ],
        mesh=mesh,
        scratch_types=[
            pltpu.VMEM((4, 128), jnp.int32),
            pltpu.VMEM((4, 128), jnp.float32),
            pltpu.VMEM((N_TOK // NW, C), jnp.float32),
            pltpu.SemaphoreType.DMA,
        ],
        compiler_params=pltpu.CompilerParams(needs_layout_passes=False),
    )(pos2d, wn2d, flat)

    grid_spec = pltpu.PrefetchScalarGridSpec(
        num_scalar_prefetch=1,
        grid=(N_TILES,),
        in_specs=[
            pl.BlockSpec((TILE, C), lambda i, te_ref: (i, 0)),
            pl.BlockSpec((1, 1, TILE), lambda i, te_ref: (i, 0, 0)),
            pl.BlockSpec((1, HID_PAD, C),
                         lambda i, te_ref: (te_ref[i], 0, 0)),
            pl.BlockSpec((1, HID_PAD, C),
                         lambda i, te_ref: (te_ref[i], 0, 0)),
            pl.BlockSpec((1, C, HID_PAD),
                         lambda i, te_ref: (te_ref[i], 0, 0)),
        ],
        out_specs=pl.BlockSpec((TILE, C), lambda i, te_ref: (i, 0)),
    )
    ex = pl.pallas_call(
        _expert_kernel,
        grid_spec=grid_spec,
        out_shape=jax.ShapeDtypeStruct((CAP, C), jnp.float32),
        compiler_params=pltpu.CompilerParams(
            dimension_semantics=("arbitrary",),
        ),
    )(te, xs, wrow.reshape(N_TILES, 1, TILE), W1p, W3p, W2p)

    out = pl.kernel(
        _combine_kernel,
        out_type=jax.ShapeDtypeStruct((N_TOK, C), jnp.float32),
        mesh=mesh,
        scratch_types=[
            pltpu.VMEM((128,), jnp.int32),
            pltpu.VMEM((128,), jnp.int32),
            pltpu.VMEM((128, C), jnp.float32),
            pltpu.VMEM((128, C), jnp.float32),
            pltpu.SemaphoreType.DMA,
            pltpu.SemaphoreType.DMA,
        ],
        compiler_params=pltpu.CompilerParams(needs_layout_passes=False),
    )(pos, ex)

    return out.reshape(B, T, C)
